# Initial kernel scaffold; baseline (speedup 1.0000x reference)
import jax
import jax.numpy as jnp
from jax.experimental import pallas as pl


def kernel(user_id, user_history, user_dense, product_id, product_category, product_dense, user_id_table, user_hist_table, product_id_table, product_cat_table, W_dense, b_dense):
    raise NotImplementedError("write your pallas kernel here")


# speedup vs baseline: 6.9332x; 6.9332x over previous
import functools

import jax
import jax.numpy as jnp
from jax import lax
from jax.experimental import pallas as pl
from jax.experimental.pallas import tpu as pltpu
from jax.experimental.pallas import tpu_sc as plsc

B = 16384
HIST = 200
D = 32
D_CAT = 16

NC = 2
NS = 16
NW = NC * NS
BPW = B // NW
CS = 8
NCHUNK = BPW // CS
SW = 100
NSTREAM = CS * HIST // SW
SB = 128
NSB = BPW // SB


def _sc_body(uh_ref, uid_ref, pid_ref, pcat_ref,
             hist_tab, uid_tab, pid_tab, pcat_tab,
             hist_out, u1_out, p1_out, p2_out,
             hidx, hrows, accbuf, sidx, srows, srows16, sem):
    wid = lax.axis_index("s") * NC + lax.axis_index("c")
    base = wid * BPW

    for idx_r, tab, out in ((uid_ref, uid_tab, u1_out), (pid_ref, pid_tab, p1_out)):
        pltpu.sync_copy(idx_r.at[wid], sidx)
        for j in range(NSB):
            pltpu.async_copy(tab.at[sidx.at[j]], srows, sem).wait()
            pltpu.sync_copy(srows, out.at[pl.ds(base + j * SB, SB)])
    pltpu.sync_copy(pcat_ref.at[wid], sidx)
    for j in range(NSB):
        pltpu.async_copy(pcat_tab.at[sidx.at[j]], srows16, sem).wait()
        pltpu.sync_copy(srows16, p2_out.at[pl.ds(base + j * SB, SB)])

    def chunk(g, carry):
        pltpu.sync_copy(uh_ref.at[wid, g], hidx)
        handles = [
            pltpu.async_copy(hist_tab.at[hidx.at[j]],
                             hrows.at[pl.ds(j * SW, SW)], sem)
            for j in range(NSTREAM)
        ]
        for h in handles:
            h.wait()
        for s in range(CS):
            rbase = s * HIST

            def acc_body(k, carry2):
                a0, a1 = carry2
                for u in range(8):
                    r = rbase + k * 8 + u
                    a0 = a0 + hrows[r, pl.ds(0, 16)]
                    a1 = a1 + hrows[r, pl.ds(16, 16)]
                return a0, a1

            z = jnp.zeros((16,), jnp.float32)
            a0, a1 = lax.fori_loop(0, HIST // 8, acc_body, (z, z))
            accbuf[s, pl.ds(0, 16)] = a0
            accbuf[s, pl.ds(16, 16)] = a1
        pltpu.sync_copy(accbuf, hist_out.at[pl.ds(base + g * CS, CS)])
        return carry

    lax.fori_loop(0, NCHUNK, chunk, 0)


def _dense_mm(x_ref, w_ref, b_ref, o_ref):
    o_ref[...] = (jnp.dot(x_ref[...], w_ref[...],
                          preferred_element_type=jnp.float32) + b_ref[...])


def kernel(user_id, user_history, user_dense, product_id, product_category,
           product_dense, user_id_table, user_hist_table, product_id_table,
           product_cat_table, W_dense, b_dense):
    uh_r = user_history.reshape(NW, NCHUNK, NSTREAM, SW).astype(jnp.int32)
    uid_r = user_id.reshape(NW, NSB, SB).astype(jnp.int32)
    pid_r = product_id.reshape(NW, NSB, SB).astype(jnp.int32)
    pcat_r = product_category.reshape(NW, NSB, SB).astype(jnp.int32)

    mesh = plsc.VectorSubcoreMesh(core_axis_name="c", subcore_axis_name="s")
    sc = functools.partial(
        pl.kernel, mesh=mesh,
        compiler_params=pltpu.CompilerParams(use_tc_tiling_on_sc=False),
        out_type=[
            jax.ShapeDtypeStruct((B, D), jnp.float32),
            jax.ShapeDtypeStruct((B, D), jnp.float32),
            jax.ShapeDtypeStruct((B, D), jnp.float32),
            jax.ShapeDtypeStruct((B, D_CAT), jnp.float32),
        ],
        scratch_types=[
            pltpu.VMEM((NSTREAM, SW), jnp.int32),
            pltpu.VMEM((CS * HIST, D), jnp.float32),
            pltpu.VMEM((CS, D), jnp.float32),
            pltpu.VMEM((NSB, SB), jnp.int32),
            pltpu.VMEM((SB, D), jnp.float32),
            pltpu.VMEM((SB, D_CAT), jnp.float32),
            pltpu.SemaphoreType.DMA,
        ],
    )(_sc_body)
    hist_sum, u1, p1, p2 = sc(uh_r, uid_r, pid_r, pcat_r,
                              user_hist_table, user_id_table,
                              product_id_table, product_cat_table)

    p3 = pl.pallas_call(
        _dense_mm,
        grid=(8,),
        in_specs=[
            pl.BlockSpec((B // 8, 64), lambda i: (i, 0)),
            pl.BlockSpec((64, D), lambda i: (0, 0)),
            pl.BlockSpec((1, D), lambda i: (0, 0)),
        ],
        out_specs=pl.BlockSpec((B // 8, D), lambda i: (i, 0)),
        out_shape=jax.ShapeDtypeStruct((B, D), jnp.float32),
    )(product_dense, W_dense, b_dense.reshape(1, D))

    user_out = jnp.concatenate([u1, hist_sum, user_dense], axis=-1)
    product_out = jnp.concatenate([p1, p2, p3], axis=-1)
    return (user_out, product_out)


# speedup vs baseline: 6.9647x; 1.0045x over previous
import functools

import jax
import jax.numpy as jnp
from jax import lax
from jax.experimental import pallas as pl
from jax.experimental.pallas import tpu as pltpu
from jax.experimental.pallas import tpu_sc as plsc

B = 16384
HIST = 200
D = 32
D_CAT = 16

NC = 2
NS = 16
NW = NC * NS
BPW = B // NW
CS = 8
NCHUNK = BPW // CS
SW = CS * HIST
NSTREAM = CS * HIST // SW
SB = 128
NSB = BPW // SB


def _sc_body(uh_ref, uid_ref, pid_ref, pcat_ref,
             hist_tab, uid_tab, pid_tab, pcat_tab,
             hist_out, u1_out, p1_out, p2_out,
             hidx, hrows, accbuf, sidx, srows, srows16, sem):
    wid = lax.axis_index("s") * NC + lax.axis_index("c")
    base = wid * BPW

    for idx_r, tab, out in ((uid_ref, uid_tab, u1_out), (pid_ref, pid_tab, p1_out)):
        pltpu.sync_copy(idx_r.at[wid], sidx)
        for j in range(NSB):
            pltpu.async_copy(tab.at[sidx.at[j]], srows, sem).wait()
            pltpu.sync_copy(srows, out.at[pl.ds(base + j * SB, SB)])
    pltpu.sync_copy(pcat_ref.at[wid], sidx)
    for j in range(NSB):
        pltpu.async_copy(pcat_tab.at[sidx.at[j]], srows16, sem).wait()
        pltpu.sync_copy(srows16, p2_out.at[pl.ds(base + j * SB, SB)])

    def chunk(g, carry):
        pltpu.sync_copy(uh_ref.at[wid, g], hidx)
        handles = [
            pltpu.async_copy(hist_tab.at[hidx.at[j]],
                             hrows.at[pl.ds(j * SW, SW)], sem)
            for j in range(NSTREAM)
        ]
        for h in handles:
            h.wait()
        for s in range(CS):
            rbase = s * HIST

            def acc_body(k, carry2):
                a0, a1 = carry2
                for u in range(8):
                    r = rbase + k * 8 + u
                    a0 = a0 + hrows[r, pl.ds(0, 16)]
                    a1 = a1 + hrows[r, pl.ds(16, 16)]
                return a0, a1

            z = jnp.zeros((16,), jnp.float32)
            a0, a1 = lax.fori_loop(0, HIST // 8, acc_body, (z, z))
            accbuf[s, pl.ds(0, 16)] = a0
            accbuf[s, pl.ds(16, 16)] = a1
        pltpu.sync_copy(accbuf, hist_out.at[pl.ds(base + g * CS, CS)])
        return carry

    lax.fori_loop(0, NCHUNK, chunk, 0)


def _dense_mm(x_ref, w_ref, b_ref, o_ref):
    o_ref[...] = (jnp.dot(x_ref[...], w_ref[...],
                          preferred_element_type=jnp.float32) + b_ref[...])


def kernel(user_id, user_history, user_dense, product_id, product_category,
           product_dense, user_id_table, user_hist_table, product_id_table,
           product_cat_table, W_dense, b_dense):
    uh_r = user_history.reshape(NW, NCHUNK, NSTREAM, SW).astype(jnp.int32)
    uid_r = user_id.reshape(NW, NSB, SB).astype(jnp.int32)
    pid_r = product_id.reshape(NW, NSB, SB).astype(jnp.int32)
    pcat_r = product_category.reshape(NW, NSB, SB).astype(jnp.int32)

    mesh = plsc.VectorSubcoreMesh(core_axis_name="c", subcore_axis_name="s")
    sc = functools.partial(
        pl.kernel, mesh=mesh,
        compiler_params=pltpu.CompilerParams(use_tc_tiling_on_sc=False),
        out_type=[
            jax.ShapeDtypeStruct((B, D), jnp.float32),
            jax.ShapeDtypeStruct((B, D), jnp.float32),
            jax.ShapeDtypeStruct((B, D), jnp.float32),
            jax.ShapeDtypeStruct((B, D_CAT), jnp.float32),
        ],
        scratch_types=[
            pltpu.VMEM((NSTREAM, SW), jnp.int32),
            pltpu.VMEM((CS * HIST, D), jnp.float32),
            pltpu.VMEM((CS, D), jnp.float32),
            pltpu.VMEM((NSB, SB), jnp.int32),
            pltpu.VMEM((SB, D), jnp.float32),
            pltpu.VMEM((SB, D_CAT), jnp.float32),
            pltpu.SemaphoreType.DMA,
        ],
    )(_sc_body)
    hist_sum, u1, p1, p2 = sc(uh_r, uid_r, pid_r, pcat_r,
                              user_hist_table, user_id_table,
                              product_id_table, product_cat_table)

    p3 = pl.pallas_call(
        _dense_mm,
        grid=(8,),
        in_specs=[
            pl.BlockSpec((B // 8, 64), lambda i: (i, 0)),
            pl.BlockSpec((64, D), lambda i: (0, 0)),
            pl.BlockSpec((1, D), lambda i: (0, 0)),
        ],
        out_specs=pl.BlockSpec((B // 8, D), lambda i: (i, 0)),
        out_shape=jax.ShapeDtypeStruct((B, D), jnp.float32),
    )(product_dense, W_dense, b_dense.reshape(1, D))

    user_out = jnp.concatenate([u1, hist_sum, user_dense], axis=-1)
    product_out = jnp.concatenate([p1, p2, p3], axis=-1)
    return (user_out, product_out)


# speedup vs baseline: 7.0111x; 1.0067x over previous
import functools

import jax
import jax.numpy as jnp
from jax import lax
from jax.experimental import pallas as pl
from jax.experimental.pallas import tpu as pltpu
from jax.experimental.pallas import tpu_sc as plsc

B = 16384
HIST = 200
D = 32
D_CAT = 16

NC = 2
NS = 16
NW = NC * NS
BPW = B // NW
CS = 8
NCHUNK = BPW // CS


def _sc_body(uh_ref, uid_ref, pid_ref, pcat_ref,
             hist_tab, uid_tab, pid_tab, pcat_tab,
             hist_out, u1_out, p1_out, p2_out,
             hidx, hrows, accbuf, sidx, srows, srows16, sem):
    wid = lax.axis_index("s") * NC + lax.axis_index("c")
    base = wid * BPW

    for idx_r, tab, out, rows in ((uid_ref, uid_tab, u1_out, srows),
                                  (pid_ref, pid_tab, p1_out, srows),
                                  (pcat_ref, pcat_tab, p2_out, srows16)):
        pltpu.sync_copy(idx_r.at[pl.ds(base, BPW)], sidx)
        pltpu.async_copy(tab.at[sidx], rows, sem).wait()
        pltpu.sync_copy(rows, out.at[pl.ds(base, BPW)])

    def chunk(g, carry):
        row0 = base + g * CS
        pltpu.sync_copy(uh_ref.at[pl.ds(row0, CS)], hidx)
        handles = [
            pltpu.async_copy(hist_tab.at[hidx.at[s]],
                             hrows.at[pl.ds(s * HIST, HIST)], sem)
            for s in range(CS)
        ]
        for h in handles:
            h.wait()
        for s in range(CS):
            rbase = s * HIST

            def acc_body(k, carry2):
                a0, a1 = carry2
                for u in range(8):
                    r = rbase + k * 8 + u
                    a0 = a0 + hrows[r, pl.ds(0, 16)]
                    a1 = a1 + hrows[r, pl.ds(16, 16)]
                return a0, a1

            z = jnp.zeros((16,), jnp.float32)
            a0, a1 = lax.fori_loop(0, HIST // 8, acc_body, (z, z))
            accbuf[s, pl.ds(0, 16)] = a0
            accbuf[s, pl.ds(16, 16)] = a1
        pltpu.sync_copy(accbuf, hist_out.at[pl.ds(row0, CS)])
        return carry

    lax.fori_loop(0, NCHUNK, chunk, 0)


def _dense_mm(x_ref, w_ref, b_ref, o_ref):
    o_ref[...] = (jnp.dot(x_ref[...], w_ref[...],
                          preferred_element_type=jnp.float32) + b_ref[...])


def kernel(user_id, user_history, user_dense, product_id, product_category,
           product_dense, user_id_table, user_hist_table, product_id_table,
           product_cat_table, W_dense, b_dense):
    mesh = plsc.VectorSubcoreMesh(core_axis_name="c", subcore_axis_name="s")
    sc = functools.partial(
        pl.kernel, mesh=mesh,
        compiler_params=pltpu.CompilerParams(use_tc_tiling_on_sc=False),
        out_type=[
            jax.ShapeDtypeStruct((B, D), jnp.float32),
            jax.ShapeDtypeStruct((B, D), jnp.float32),
            jax.ShapeDtypeStruct((B, D), jnp.float32),
            jax.ShapeDtypeStruct((B, D_CAT), jnp.float32),
        ],
        scratch_types=[
            pltpu.VMEM((CS, HIST), jnp.int32),
            pltpu.VMEM((CS * HIST, D), jnp.float32),
            pltpu.VMEM((CS, D), jnp.float32),
            pltpu.VMEM((BPW,), jnp.int32),
            pltpu.VMEM((BPW, D), jnp.float32),
            pltpu.VMEM((BPW, D_CAT), jnp.float32),
            pltpu.SemaphoreType.DMA,
        ],
    )(_sc_body)
    hist_sum, u1, p1, p2 = sc(user_history, user_id, product_id,
                              product_category,
                              user_hist_table, user_id_table,
                              product_id_table, product_cat_table)

    p3 = pl.pallas_call(
        _dense_mm,
        grid=(8,),
        in_specs=[
            pl.BlockSpec((B // 8, 64), lambda i: (i, 0)),
            pl.BlockSpec((64, D), lambda i: (0, 0)),
            pl.BlockSpec((1, D), lambda i: (0, 0)),
        ],
        out_specs=pl.BlockSpec((B // 8, D), lambda i: (i, 0)),
        out_shape=jax.ShapeDtypeStruct((B, D), jnp.float32),
    )(product_dense, W_dense, b_dense.reshape(1, D))

    user_out = jnp.concatenate([u1, hist_sum, user_dense], axis=-1)
    product_out = jnp.concatenate([p1, p2, p3], axis=-1)
    return (user_out, product_out)


# speedup vs baseline: 7.6568x; 1.0921x over previous
import functools

import jax
import jax.numpy as jnp
from jax import lax
from jax.experimental import pallas as pl
from jax.experimental.pallas import tpu as pltpu
from jax.experimental.pallas import tpu_sc as plsc

B = 16384
HIST = 200
D = 32
D_CAT = 16

NC = 2
NS = 16
NW = NC * NS
BPW = B // NW
CS = 8
NCHUNK = BPW // CS


def _sc_body(uh_ref, uid_ref, pid_ref, pcat_ref,
             hist_tab, uid_tab, pid_tab, pcat_tab,
             hist_out, u1_out, p1_out, p2_out,
             hidx_a, hidx_b, hrows_a, hrows_b, accbuf,
             sidx, srows, srows16, sem_a, sem_b, sem_s):
    wid = lax.axis_index("s") * NC + lax.axis_index("c")
    base = wid * BPW

    def fire(hidx, hrows, sem):
        for s in range(CS):
            pltpu.async_copy(hist_tab.at[hidx.at[s]],
                             hrows.at[pl.ds(s * HIST, HIST)], sem)

    def drain(hrows, sem):
        pltpu.make_async_copy(hist_tab.at[pl.ds(0, CS * HIST)], hrows,
                              sem).wait()

    def reduce(g, hrows):
        for s in range(CS):
            rbase = s * HIST

            def acc_body(k, carry2):
                a0, a1 = carry2
                for u in range(8):
                    r = rbase + k * 8 + u
                    a0 = a0 + hrows[r, pl.ds(0, 16)]
                    a1 = a1 + hrows[r, pl.ds(16, 16)]
                return a0, a1

            z = jnp.zeros((16,), jnp.float32)
            a0, a1 = lax.fori_loop(0, HIST // 8, acc_body, (z, z))
            accbuf[s, pl.ds(0, 16)] = a0
            accbuf[s, pl.ds(16, 16)] = a1
        pltpu.sync_copy(accbuf, hist_out.at[pl.ds(base + g * CS, CS)])

    pltpu.sync_copy(uh_ref.at[pl.ds(base, CS)], hidx_a)
    fire(hidx_a, hrows_a, sem_a)

    def pair(k, carry):
        g0 = 2 * k
        pltpu.sync_copy(uh_ref.at[pl.ds(base + (g0 + 1) * CS, CS)], hidx_b)
        fire(hidx_b, hrows_b, sem_b)
        drain(hrows_a, sem_a)
        reduce(g0, hrows_a)

        @pl.when(k < NCHUNK // 2 - 1)
        def _():
            pltpu.sync_copy(uh_ref.at[pl.ds(base + (g0 + 2) * CS, CS)],
                            hidx_a)
            fire(hidx_a, hrows_a, sem_a)

        drain(hrows_b, sem_b)
        reduce(g0 + 1, hrows_b)
        return carry

    lax.fori_loop(0, NCHUNK // 2, pair, 0)

    for idx_r, tab, out, rows, w in ((uid_ref, uid_tab, u1_out, srows, D),
                                     (pid_ref, pid_tab, p1_out, srows, D),
                                     (pcat_ref, pcat_tab, p2_out, srows16,
                                      D_CAT)):
        for h in range(2):
            pltpu.sync_copy(idx_r.at[pl.ds(base + h * (BPW // 2), BPW // 2)],
                            sidx)
            pltpu.async_copy(tab.at[sidx], rows, sem_s).wait()
            pltpu.sync_copy(rows,
                            out.at[pl.ds(base + h * (BPW // 2), BPW // 2)])


def _dense_mm(x_ref, w_ref, b_ref, o_ref):
    o_ref[...] = (jnp.dot(x_ref[...], w_ref[...],
                          preferred_element_type=jnp.float32) + b_ref[...])


def kernel(user_id, user_history, user_dense, product_id, product_category,
           product_dense, user_id_table, user_hist_table, product_id_table,
           product_cat_table, W_dense, b_dense):
    mesh = plsc.VectorSubcoreMesh(core_axis_name="c", subcore_axis_name="s")
    sc = functools.partial(
        pl.kernel, mesh=mesh,
        compiler_params=pltpu.CompilerParams(use_tc_tiling_on_sc=False),
        out_type=[
            jax.ShapeDtypeStruct((B, D), jnp.float32),
            jax.ShapeDtypeStruct((B, D), jnp.float32),
            jax.ShapeDtypeStruct((B, D), jnp.float32),
            jax.ShapeDtypeStruct((B, D_CAT), jnp.float32),
        ],
        scratch_types=[
            pltpu.VMEM((CS, HIST), jnp.int32),
            pltpu.VMEM((CS, HIST), jnp.int32),
            pltpu.VMEM((CS * HIST, D), jnp.float32),
            pltpu.VMEM((CS * HIST, D), jnp.float32),
            pltpu.VMEM((CS, D), jnp.float32),
            pltpu.VMEM((BPW // 2,), jnp.int32),
            pltpu.VMEM((BPW // 2, D), jnp.float32),
            pltpu.VMEM((BPW // 2, D_CAT), jnp.float32),
            pltpu.SemaphoreType.DMA,
            pltpu.SemaphoreType.DMA,
            pltpu.SemaphoreType.DMA,
        ],
    )(_sc_body)
    hist_sum, u1, p1, p2 = sc(user_history, user_id, product_id,
                              product_category,
                              user_hist_table, user_id_table,
                              product_id_table, product_cat_table)

    p3 = pl.pallas_call(
        _dense_mm,
        grid=(8,),
        in_specs=[
            pl.BlockSpec((B // 8, 64), lambda i: (i, 0)),
            pl.BlockSpec((64, D), lambda i: (0, 0)),
            pl.BlockSpec((1, D), lambda i: (0, 0)),
        ],
        out_specs=pl.BlockSpec((B // 8, D), lambda i: (i, 0)),
        out_shape=jax.ShapeDtypeStruct((B, D), jnp.float32),
    )(product_dense, W_dense, b_dense.reshape(1, D))

    user_out = jnp.concatenate([u1, hist_sum, user_dense], axis=-1)
    product_out = jnp.concatenate([p1, p2, p3], axis=-1)
    return (user_out, product_out)


# speedup vs baseline: 7.6581x; 1.0002x over previous
import functools

import jax
import jax.numpy as jnp
from jax import lax
from jax.experimental import pallas as pl
from jax.experimental.pallas import tpu as pltpu
from jax.experimental.pallas import tpu_sc as plsc

B = 16384
HIST = 200
D = 32
D_CAT = 16

NC = 2
NS = 16
NW = NC * NS
BPW = B // NW
CS = 8
NCHUNK = BPW // CS


def _sc_body(uh_ref, pcat_ref, hist_tab, pcat_tab,
             hist_out, p2_out,
             hidx_a, hidx_b, hrows_a, hrows_b, accbuf,
             sidx, srows16, sem_a, sem_b, sem_s):
    wid = lax.axis_index("s") * NC + lax.axis_index("c")
    base = wid * BPW

    def fire(hidx, hrows, sem):
        for s in range(CS):
            pltpu.async_copy(hist_tab.at[hidx.at[s]],
                             hrows.at[pl.ds(s * HIST, HIST)], sem)

    def drain(hrows, sem):
        pltpu.make_async_copy(hist_tab.at[pl.ds(0, CS * HIST)], hrows,
                              sem).wait()

    def reduce(g, hrows):
        for s in range(CS):
            rbase = s * HIST

            def acc_body(k, carry2):
                a0, a1 = carry2
                for u in range(8):
                    r = rbase + k * 8 + u
                    a0 = a0 + hrows[r, pl.ds(0, 16)]
                    a1 = a1 + hrows[r, pl.ds(16, 16)]
                return a0, a1

            z = jnp.zeros((16,), jnp.float32)
            a0, a1 = lax.fori_loop(0, HIST // 8, acc_body, (z, z))
            accbuf[s, pl.ds(0, 16)] = a0
            accbuf[s, pl.ds(16, 16)] = a1
        pltpu.sync_copy(accbuf, hist_out.at[pl.ds(base + g * CS, CS)])

    pltpu.sync_copy(uh_ref.at[pl.ds(base, CS)], hidx_a)
    fire(hidx_a, hrows_a, sem_a)

    def pair(k, carry):
        g0 = 2 * k
        pltpu.sync_copy(uh_ref.at[pl.ds(base + (g0 + 1) * CS, CS)], hidx_b)
        fire(hidx_b, hrows_b, sem_b)
        drain(hrows_a, sem_a)
        reduce(g0, hrows_a)

        @pl.when(k < NCHUNK // 2 - 1)
        def _():
            pltpu.sync_copy(uh_ref.at[pl.ds(base + (g0 + 2) * CS, CS)],
                            hidx_a)
            fire(hidx_a, hrows_a, sem_a)

        drain(hrows_b, sem_b)
        reduce(g0 + 1, hrows_b)
        return carry

    lax.fori_loop(0, NCHUNK // 2, pair, 0)

    for h in range(2):
        pltpu.sync_copy(pcat_ref.at[pl.ds(base + h * (BPW // 2), BPW // 2)],
                        sidx)
        pltpu.async_copy(pcat_tab.at[sidx], srows16, sem_s).wait()
        pltpu.sync_copy(srows16,
                        p2_out.at[pl.ds(base + h * (BPW // 2), BPW // 2)])


def _sc_lookup_body(uid_ref, pid_ref, utab_ref, ptab_ref,
                    u1_out, p1_out, sidx, ridx, rows, obuf, sem):
    wid = lax.axis_index("s") * NC + lax.axis_index("c")
    base = wid * BPW
    iota = lax.broadcasted_iota(jnp.int32, (16,), 0)
    HB = BPW // 2
    for idx_r, tab, out in ((uid_ref, utab_ref, u1_out),
                            (pid_ref, ptab_ref, p1_out)):
        for h in range(2):
            b0 = base + h * HB
            pltpu.sync_copy(idx_r.at[pl.ds(b0, HB)], sidx)

            def prep(i, c):
                sv = sidx[pl.ds(i * 16, 16)]
                ridx[pl.ds(i * 16, 16)] = sv >> 2
                return c

            lax.fori_loop(0, HB // 16, prep, 0)
            pltpu.async_copy(tab.at[ridx], rows, sem).wait()

            def ext(i, c):
                sv = sidx[pl.ds(i * 16, 16)]
                cv = (sv & 3) * 32
                row_v = i * 16 + iota
                dbase = row_v * D
                for j in range(D):
                    vals = plsc.load_gather(rows, [row_v, cv + j])
                    plsc.store_scatter(obuf, [dbase + j], vals)
                return c

            lax.fori_loop(0, HB // 16, ext, 0)
            pltpu.sync_copy(obuf, out.at[pl.ds(b0 * D, HB * D)])


def _dense_mm(x_ref, w_ref, b_ref, o_ref):
    o_ref[...] = (jnp.dot(x_ref[...], w_ref[...],
                          preferred_element_type=jnp.float32) + b_ref[...])


def kernel(user_id, user_history, user_dense, product_id, product_category,
           product_dense, user_id_table, user_hist_table, product_id_table,
           product_cat_table, W_dense, b_dense):
    mesh = plsc.VectorSubcoreMesh(core_axis_name="c", subcore_axis_name="s")
    sc = functools.partial(
        pl.kernel, mesh=mesh,
        compiler_params=pltpu.CompilerParams(use_tc_tiling_on_sc=False),
        out_type=[
            jax.ShapeDtypeStruct((B, D), jnp.float32),
            jax.ShapeDtypeStruct((B, D_CAT), jnp.float32),
        ],
        scratch_types=[
            pltpu.VMEM((CS, HIST), jnp.int32),
            pltpu.VMEM((CS, HIST), jnp.int32),
            pltpu.VMEM((CS * HIST, D), jnp.float32),
            pltpu.VMEM((CS * HIST, D), jnp.float32),
            pltpu.VMEM((CS, D), jnp.float32),
            pltpu.VMEM((BPW // 2,), jnp.int32),
            pltpu.VMEM((BPW // 2, D_CAT), jnp.float32),
            pltpu.SemaphoreType.DMA,
            pltpu.SemaphoreType.DMA,
            pltpu.SemaphoreType.DMA,
        ],
    )(_sc_body)
    hist_sum, p2 = sc(user_history, product_category,
                      user_hist_table, product_cat_table)

    sc_lk = functools.partial(
        pl.kernel, mesh=mesh,
        compiler_params=pltpu.CompilerParams(use_tc_tiling_on_sc=True,
                                             needs_layout_passes=False),
        out_type=[
            jax.ShapeDtypeStruct((B * D,), jnp.float32),
            jax.ShapeDtypeStruct((B * D,), jnp.float32),
        ],
        scratch_types=[
            pltpu.VMEM((BPW // 2,), jnp.int32),
            pltpu.VMEM((BPW // 2,), jnp.int32),
            pltpu.VMEM((BPW // 2, 128), jnp.float32),
            pltpu.VMEM((BPW // 2 * D,), jnp.float32),
            pltpu.SemaphoreType.DMA,
        ],
    )(_sc_lookup_body)
    u1f, p1f = sc_lk(user_id, product_id,
                     user_id_table.reshape(250000, 128),
                     product_id_table.reshape(250000, 128))
    u1 = u1f.reshape(B, D)
    p1 = p1f.reshape(B, D)

    p3 = pl.pallas_call(
        _dense_mm,
        grid=(8,),
        in_specs=[
            pl.BlockSpec((B // 8, 64), lambda i: (i, 0)),
            pl.BlockSpec((64, D), lambda i: (0, 0)),
            pl.BlockSpec((1, D), lambda i: (0, 0)),
        ],
        out_specs=pl.BlockSpec((B // 8, D), lambda i: (i, 0)),
        out_shape=jax.ShapeDtypeStruct((B, D), jnp.float32),
    )(product_dense, W_dense, b_dense.reshape(1, D))

    user_out = jnp.concatenate([u1, hist_sum, user_dense], axis=-1)
    product_out = jnp.concatenate([p1, p2, p3], axis=-1)
    return (user_out, product_out)
